# baseline (speedup 1.0000x reference)
import functools

import jax
import jax.numpy as jnp
from jax import lax
from jax.experimental import pallas as pl
from jax.experimental.pallas import tpu as pltpu
from jax.experimental.pallas import tpu_sc as plsc

N = 10000
E = 160000
D = 248
DP = 256
H = DP // 2
NNZ_PAD = 3840

NC, NS = 2, 16
CHUNK_E = 128
NCH = 80
EP = NS * NCH * CHUNK_E
NP = 10112
STRIPE = NP // NS


def _segment_sum_sc(feat2, src2, tgt3):
    mesh = plsc.VectorSubcoreMesh(core_axis_name="c", subcore_axis_name="s")

    @functools.partial(
        pl.kernel,
        out_type=jax.ShapeDtypeStruct((NC * NP, H), jnp.float32),
        mesh=mesh,
        scratch_types=[
            pltpu.VMEM_SHARED((NP, H), jnp.float32),
            pltpu.VMEM((NCH // 2, CHUNK_E), jnp.int32),
            pltpu.VMEM((NCH // 2, CHUNK_E), jnp.int32),
            pltpu.VMEM((CHUNK_E, H), jnp.float32),
            pltpu.VMEM((CHUNK_E, H), jnp.float32),
            pltpu.SemaphoreType.DMA,
            pltpu.SemaphoreType.DMA,
        ],
    )
    def seg(feat2_hbm, src2_hbm, tgt3_hbm, out_hbm, acc_sh, src_v, tgt_v,
            rows_v, rows_b, sem, sem_b):
        c = lax.axis_index("c")
        s = lax.axis_index("s")
        w = c * NS + s

        def zrow(r, carry):
            for q in range(H // 16):
                rows_v[r, pl.ds(q * 16, 16)] = jnp.zeros((16,), jnp.float32)
            return carry
        lax.fori_loop(0, CHUNK_E, zrow, 0)
        for t in range(STRIPE // CHUNK_E):
            pltpu.sync_copy(
                rows_v, acc_sh.at[pl.ds(s * STRIPE + t * CHUNK_E, CHUNK_E)])
        rem = STRIPE % CHUNK_E
        pltpu.sync_copy(
            rows_v.at[pl.ds(0, rem)],
            acc_sh.at[pl.ds(s * STRIPE + STRIPE - rem, rem)])

        plsc.subcore_barrier()

        def gather(j, buf, gsem):
            pltpu.async_copy(
                feat2_hbm.at[plsc.Indices(src_v.at[j], ignored_value=-1)],
                buf, gsem)

        def gwait(j, buf, gsem):
            pltpu.make_async_copy(
                feat2_hbm.at[plsc.Indices(src_v.at[j], ignored_value=-1)],
                buf, gsem).wait()

        def scat(j, buf):
            pltpu.sync_copy(
                buf, acc_sh.at[plsc.Indices(tgt_v.at[j], ignored_value=-1)],
                add=True)

        nhalf = NCH // 2
        npair = nhalf // 2
        for p in range(2):
            pltpu.sync_copy(src2_hbm.at[w, pl.ds(p * nhalf, nhalf)], src_v)
            pltpu.sync_copy(tgt3_hbm.at[s, pl.ds(p * nhalf, nhalf)], tgt_v)
            gather(0, rows_v, sem)

            def pair(g, carry):
                j0 = 2 * g
                gather(j0 + 1, rows_b, sem_b)
                gwait(j0, rows_v, sem)
                scat(j0, rows_v)

                @pl.when(g + 1 < npair)
                def _():
                    gather(j0 + 2, rows_v, sem)

                gwait(j0 + 1, rows_b, sem_b)
                scat(j0 + 1, rows_b)
                return carry
            lax.fori_loop(0, npair, pair, 0)

        plsc.subcore_barrier()

        base = c * NP + s * STRIPE
        for t in range(STRIPE // CHUNK_E):
            pltpu.sync_copy(
                acc_sh.at[pl.ds(s * STRIPE + t * CHUNK_E, CHUNK_E)], rows_v)
            pltpu.sync_copy(
                rows_v, out_hbm.at[pl.ds(base + t * CHUNK_E, CHUNK_E)])
        pltpu.sync_copy(
            acc_sh.at[pl.ds(s * STRIPE + STRIPE - rem, rem)],
            rows_v.at[pl.ds(0, rem)])
        pltpu.sync_copy(
            rows_v.at[pl.ds(0, rem)],
            out_hbm.at[pl.ds(base + STRIPE - rem, rem)])

    return seg(feat2, src2, tgt3)


def _bracket_update_tc(x, seg3, gi, gj, hk):
    BR = 2000

    KC = NNZ_PAD // 6

    def body(x_ref, sl_ref, sr_ref, gi_ref, gj_ref, hk_ref, o_ref):
        s = jnp.concatenate(
            [sl_ref[0], sr_ref[0]], axis=1).astype(jnp.bfloat16)
        xb = x_ref[...].astype(jnp.bfloat16)
        acc = x_ref[...]
        for kc in range(NNZ_PAD // KC):
            sl_k = pl.ds(kc * KC, KC)
            a = jnp.dot(s, gi_ref[:, sl_k],
                        preferred_element_type=jnp.float32)
            b = jnp.dot(xb, gj_ref[:, sl_k],
                        preferred_element_type=jnp.float32)
            t = (a * b).astype(jnp.bfloat16)
            acc += jnp.dot(t, hk_ref[sl_k, :],
                           preferred_element_type=jnp.float32)
        o_ref[...] = acc

    return pl.pallas_call(
        body,
        grid=(N // BR,),
        in_specs=[
            pl.BlockSpec((BR, D), lambda i: (i, 0)),
            pl.BlockSpec((1, BR, H), lambda i: (0, i, 0)),
            pl.BlockSpec((1, BR, H), lambda i: (1, i, 0)),
            pl.BlockSpec((DP, NNZ_PAD), lambda i: (0, 0)),
            pl.BlockSpec((D, NNZ_PAD), lambda i: (0, 0)),
            pl.BlockSpec((NNZ_PAD, D), lambda i: (0, 0)),
        ],
        out_specs=pl.BlockSpec((BR, D), lambda i: (i, 0)),
        out_shape=jax.ShapeDtypeStruct((N, D), jnp.float32),
    )(x, seg3, seg3, gi, gj, hk)


def kernel(features, edge_index, f_idx, f_val, alpha_proj, alpha_bil,
           alpha_W, update_scale):
    del alpha_W, update_scale

    feat2 = jnp.concatenate(
        [features[:, :H],
         jnp.pad(features[:, H:], ((0, 0), (0, DP - D)))], axis=0)

    src = edge_index[0]
    tgt = edge_index[1]
    src2 = jnp.concatenate(
        [jnp.pad(src, (0, EP - E), constant_values=-1),
         jnp.pad(src + N, (0, EP - E), constant_values=-1)]
    ).reshape(NC * NS, NCH, CHUNK_E)
    tgt3 = jnp.pad(tgt, (0, EP - E),
                   constant_values=-1).reshape(NS, NCH, CHUNK_E)

    seg = _segment_sum_sc(feat2, src2, tgt3)
    seg3 = seg.reshape(NC, NP, H)

    nnz = f_idx.shape[0]
    fi = jnp.pad(f_idx[:, 0], (0, NNZ_PAD - nnz))
    fj = jnp.pad(f_idx[:, 1], (0, NNZ_PAD - nnz))
    fk = jnp.pad(f_idx[:, 2], (0, NNZ_PAD - nnz))
    fv = jnp.pad(f_val, (0, NNZ_PAD - nnz))
    ar = jnp.arange(DP, dtype=f_idx.dtype)
    gi = (fi[None, :] == ar[:, None]).astype(jnp.bfloat16)
    gj = (fj[None, :] == ar[:D, None]).astype(jnp.bfloat16)
    fvs = alpha_bil * alpha_proj * fv
    hk = (jnp.where(fk[:, None] == ar[None, :D], fvs[:, None], 0.0)
          ).astype(jnp.bfloat16)

    return _bracket_update_tc(features, seg3, gi, gj, hk)


# speedup vs baseline: 1.0633x; 1.0633x over previous
import functools

import jax
import jax.numpy as jnp
from jax import lax
from jax.experimental import pallas as pl
from jax.experimental.pallas import tpu as pltpu
from jax.experimental.pallas import tpu_sc as plsc

N = 10000
E = 160000
D = 248
DP = 256
H = DP // 2
NNZ_PAD = 3840

NC, NS = 2, 16
CHUNK_E = 128
NCH = 80
EP = NS * NCH * CHUNK_E
NP = 10112
STRIPE = NP // NS


def _segment_sum_sc(feat2, src2, tgt3):
    mesh = plsc.VectorSubcoreMesh(core_axis_name="c", subcore_axis_name="s")

    @functools.partial(
        pl.kernel,
        out_type=jax.ShapeDtypeStruct((NC * NP, H), jnp.float32),
        mesh=mesh,
        scratch_types=[
            pltpu.VMEM_SHARED((NP, H), jnp.float32),
            pltpu.VMEM((NCH // 2, CHUNK_E), jnp.int32),
            pltpu.VMEM((NCH // 2, CHUNK_E), jnp.int32),
            pltpu.VMEM((CHUNK_E, H), jnp.float32),
            pltpu.VMEM((CHUNK_E, H), jnp.float32),
            pltpu.SemaphoreType.DMA,
            pltpu.SemaphoreType.DMA,
        ],
    )
    def seg(feat2_hbm, src2_hbm, tgt3_hbm, out_hbm, acc_sh, src_v, tgt_v,
            rows_v, rows_b, sem, sem_b):
        c = lax.axis_index("c")
        s = lax.axis_index("s")
        w = c * NS + s

        def zrow(r, carry):
            for q in range(H // 16):
                rows_v[r, pl.ds(q * 16, 16)] = jnp.zeros((16,), jnp.float32)
            return carry
        lax.fori_loop(0, CHUNK_E, zrow, 0)
        for t in range(STRIPE // CHUNK_E):
            pltpu.sync_copy(
                rows_v, acc_sh.at[pl.ds(s * STRIPE + t * CHUNK_E, CHUNK_E)])
        rem = STRIPE % CHUNK_E
        pltpu.sync_copy(
            rows_v.at[pl.ds(0, rem)],
            acc_sh.at[pl.ds(s * STRIPE + STRIPE - rem, rem)])

        plsc.subcore_barrier()

        def gather(j, buf, gsem):
            pltpu.async_copy(
                feat2_hbm.at[plsc.Indices(src_v.at[j], ignored_value=-1)],
                buf, gsem)

        def gwait(j, buf, gsem):
            pltpu.make_async_copy(
                feat2_hbm.at[plsc.Indices(src_v.at[j], ignored_value=-1)],
                buf, gsem).wait()

        def scat(j, buf):
            pltpu.sync_copy(
                buf, acc_sh.at[plsc.Indices(tgt_v.at[j], ignored_value=-1)],
                add=True)

        nhalf = NCH // 2
        npair = nhalf // 2
        for p in range(2):
            pltpu.sync_copy(src2_hbm.at[w, pl.ds(p * nhalf, nhalf)], src_v)
            pltpu.sync_copy(tgt3_hbm.at[s, pl.ds(p * nhalf, nhalf)], tgt_v)
            gather(0, rows_v, sem)

            def pair(g, carry):
                j0 = 2 * g
                gather(j0 + 1, rows_b, sem_b)
                gwait(j0, rows_v, sem)
                scat(j0, rows_v)

                @pl.when(g + 1 < npair)
                def _():
                    gather(j0 + 2, rows_v, sem)

                gwait(j0 + 1, rows_b, sem_b)
                scat(j0 + 1, rows_b)
                return carry
            lax.fori_loop(0, npair, pair, 0)

        plsc.subcore_barrier()

        base = c * NP + s * STRIPE
        for t in range(STRIPE // CHUNK_E):
            pltpu.sync_copy(
                acc_sh.at[pl.ds(s * STRIPE + t * CHUNK_E, CHUNK_E)], rows_v)
            pltpu.sync_copy(
                rows_v, out_hbm.at[pl.ds(base + t * CHUNK_E, CHUNK_E)])
        pltpu.sync_copy(
            acc_sh.at[pl.ds(s * STRIPE + STRIPE - rem, rem)],
            rows_v.at[pl.ds(0, rem)])
        pltpu.sync_copy(
            rows_v.at[pl.ds(0, rem)],
            out_hbm.at[pl.ds(base + STRIPE - rem, rem)])

    return seg(feat2, src2, tgt3)


def _bracket_update_tc(x, seg3, gi, gj, hk):
    BR = 2000

    KC = NNZ_PAD // 5

    def body(x_ref, sl_ref, sr_ref, gi_ref, gj_ref, hk_ref, o_ref):
        s = jnp.concatenate(
            [sl_ref[0], sr_ref[0]], axis=1).astype(jnp.bfloat16)
        xb = x_ref[...].astype(jnp.bfloat16)
        acc = x_ref[...]
        for kc in range(NNZ_PAD // KC):
            sl_k = pl.ds(kc * KC, KC)
            a = jnp.dot(s, gi_ref[:, sl_k],
                        preferred_element_type=jnp.float32)
            b = jnp.dot(xb, gj_ref[:, sl_k],
                        preferred_element_type=jnp.float32)
            t = (a * b).astype(jnp.bfloat16)
            acc += jnp.dot(t, hk_ref[sl_k, :],
                           preferred_element_type=jnp.float32)
        o_ref[...] = acc

    return pl.pallas_call(
        body,
        grid=(N // BR,),
        in_specs=[
            pl.BlockSpec((BR, D), lambda i: (i, 0)),
            pl.BlockSpec((1, BR, H), lambda i: (0, i, 0)),
            pl.BlockSpec((1, BR, H), lambda i: (1, i, 0)),
            pl.BlockSpec((DP, NNZ_PAD), lambda i: (0, 0)),
            pl.BlockSpec((D, NNZ_PAD), lambda i: (0, 0)),
            pl.BlockSpec((NNZ_PAD, D), lambda i: (0, 0)),
        ],
        out_specs=pl.BlockSpec((BR, D), lambda i: (i, 0)),
        out_shape=jax.ShapeDtypeStruct((N, D), jnp.float32),
    )(x, seg3, seg3, gi, gj, hk)


def kernel(features, edge_index, f_idx, f_val, alpha_proj, alpha_bil,
           alpha_W, update_scale):
    del alpha_W, update_scale

    feat2 = jnp.concatenate(
        [features[:, :H],
         jnp.pad(features[:, H:], ((0, 0), (0, DP - D)))], axis=0)

    src = edge_index[0]
    tgt = edge_index[1]
    src2 = jnp.concatenate(
        [jnp.pad(src, (0, EP - E), constant_values=-1),
         jnp.pad(src + N, (0, EP - E), constant_values=-1)]
    ).reshape(NC * NS, NCH, CHUNK_E)
    tgt3 = jnp.pad(tgt, (0, EP - E),
                   constant_values=-1).reshape(NS, NCH, CHUNK_E)

    seg = _segment_sum_sc(feat2, src2, tgt3)
    seg3 = seg.reshape(NC, NP, H)

    nnz = f_idx.shape[0]
    fi = jnp.pad(f_idx[:, 0], (0, NNZ_PAD - nnz))
    fj = jnp.pad(f_idx[:, 1], (0, NNZ_PAD - nnz))
    fk = jnp.pad(f_idx[:, 2], (0, NNZ_PAD - nnz))
    fv = jnp.pad(f_val, (0, NNZ_PAD - nnz))
    ar = jnp.arange(DP, dtype=f_idx.dtype)
    gi = (fi[None, :] == ar[:, None]).astype(jnp.bfloat16)
    gj = (fj[None, :] == ar[:D, None]).astype(jnp.bfloat16)
    fvs = alpha_bil * alpha_proj * fv
    hk = (jnp.where(fk[:, None] == ar[None, :D], fvs[:, None], 0.0)
          ).astype(jnp.bfloat16)

    return _bracket_update_tc(features, seg3, gi, gj, hk)
